# Initial kernel scaffold; baseline (speedup 1.0000x reference)
import jax
import jax.numpy as jnp
from jax.experimental import pallas as pl


def kernel(x, edge_index, batch, W1, b1, Wl, bl, Wr, Wg, att_src, att_dst, bg, W4, b4, g1, be1, g2, be2, g3, be3, g4, be4, Wfc, bfc):
    raise NotImplementedError("write your pallas kernel here")


# speedup vs baseline: 1.1058x; 1.1058x over previous
import jax
import jax.numpy as jnp
from jax import lax
from jax.experimental import pallas as pl
from jax.experimental.pallas import tpu as pltpu

_N = 10000
_G = 64
_NB = 10
_BR = _N // _NB


def _mm_body(a_ref, b_ref, o_ref):
    o_ref[...] = jnp.dot(a_ref[...], b_ref[...],
                         preferred_element_type=jnp.float32)


def _mm(a, b):
    n, k = a.shape
    m = b.shape[1]
    return pl.pallas_call(
        _mm_body,
        grid=(_NB,),
        in_specs=[pl.BlockSpec((_BR, k), lambda i: (i, 0)),
                  pl.BlockSpec((k, m), lambda i: (0, 0))],
        out_specs=pl.BlockSpec((_BR, m), lambda i: (i, 0)),
        out_shape=jax.ShapeDtypeStruct((n, m), jnp.float32),
    )(a, b)


def _mm2_body(a1_ref, b1_ref, a2_ref, b2_ref, o_ref):
    o_ref[...] = (
        jnp.dot(a1_ref[...], b1_ref[...], preferred_element_type=jnp.float32)
        + jnp.dot(a2_ref[...], b2_ref[...],
                  preferred_element_type=jnp.float32))


def _mm2(a1, b1, a2, b2):
    n, k1 = a1.shape
    k2 = a2.shape[1]
    m = b1.shape[1]
    return pl.pallas_call(
        _mm2_body,
        grid=(_NB,),
        in_specs=[pl.BlockSpec((_BR, k1), lambda i: (i, 0)),
                  pl.BlockSpec((k1, m), lambda i: (0, 0)),
                  pl.BlockSpec((_BR, k2), lambda i: (i, 0)),
                  pl.BlockSpec((k2, m), lambda i: (0, 0))],
        out_specs=pl.BlockSpec((_BR, m), lambda i: (i, 0)),
        out_shape=jax.ShapeDtypeStruct((n, m), jnp.float32),
    )(a1, b1, a2, b2)


def _rowscale_body(h_ref, r_ref, o_ref):
    o_ref[...] = h_ref[...] * r_ref[...]


def _rowscale(h, r):
    n, c = h.shape
    return pl.pallas_call(
        _rowscale_body,
        grid=(_NB,),
        in_specs=[pl.BlockSpec((_BR, c), lambda i: (i, 0)),
                  pl.BlockSpec((_BR, 1), lambda i: (i, 0))],
        out_specs=pl.BlockSpec((_BR, c), lambda i: (i, 0)),
        out_shape=jax.ShapeDtypeStruct((n, c), jnp.float32),
    )(h, r)


def _mom_body(h_ref, s1_ref, s2_ref):
    i = pl.program_id(0)
    h = h_ref[...]
    ps1 = jnp.sum(h, axis=0, keepdims=True)
    ps2 = jnp.sum(h * h, axis=0, keepdims=True)

    @pl.when(i == 0)
    def _():
        s1_ref[...] = ps1
        s2_ref[...] = ps2

    @pl.when(i > 0)
    def _():
        s1_ref[...] += ps1
        s2_ref[...] += ps2


def _bn_apply_body(h_ref, s1_ref, s2_ref, g_ref, be_ref, o_ref):
    mu = s1_ref[...] * (1.0 / _N)
    var = s2_ref[...] * (1.0 / _N) - mu * mu
    h = (h_ref[...] - mu) * lax.rsqrt(var + 1e-5) * g_ref[...] + be_ref[...]
    o_ref[...] = jnp.maximum(h, 0.0)


def _bnrelu(h, g, be):
    n, c = h.shape
    s1, s2 = pl.pallas_call(
        _mom_body,
        grid=(_NB,),
        in_specs=[pl.BlockSpec((_BR, c), lambda i: (i, 0))],
        out_specs=[pl.BlockSpec((1, c), lambda i: (0, 0)),
                   pl.BlockSpec((1, c), lambda i: (0, 0))],
        out_shape=[jax.ShapeDtypeStruct((1, c), jnp.float32),
                   jax.ShapeDtypeStruct((1, c), jnp.float32)],
    )(h)
    return pl.pallas_call(
        _bn_apply_body,
        grid=(_NB,),
        in_specs=[pl.BlockSpec((_BR, c), lambda i: (i, 0)),
                  pl.BlockSpec((1, c), lambda i: (0, 0)),
                  pl.BlockSpec((1, c), lambda i: (0, 0)),
                  pl.BlockSpec((1, c), lambda i: (0, 0)),
                  pl.BlockSpec((1, c), lambda i: (0, 0))],
        out_specs=pl.BlockSpec((_BR, c), lambda i: (i, 0)),
        out_shape=jax.ShapeDtypeStruct((n, c), jnp.float32),
    )(h, s1, s2, g[None, :], be[None, :])


def _pool_body(h_ref, batch_ref, wfc_ref, bfc_ref, o_ref, x2_s):
    h4 = h_ref[...]
    batch = batch_ref[...]
    gids = lax.broadcasted_iota(jnp.int32, (_G, _N), 0)
    onehot = (gids == batch[:, 0][None, :]).astype(jnp.float32)
    cnt = jnp.sum(onehot, axis=1, keepdims=True)
    x1 = jnp.dot(onehot, h4, preferred_element_type=jnp.float32)
    x1 = x1 / jnp.maximum(cnt, 1.0)

    def body(g, _):
        mask = batch == g
        row = jnp.max(jnp.where(mask, h4, -jnp.inf), axis=0, keepdims=True)
        x2_s[pl.ds(g, 1), :] = row
        return 0

    lax.fori_loop(0, _G, body, 0)
    z = jnp.concatenate([x1, x2_s[...]], axis=1)
    o_ref[...] = jnp.dot(z, wfc_ref[...],
                         preferred_element_type=jnp.float32) + bfc_ref[...]


def kernel(x, edge_index, batch, W1, b1, Wl, bl, Wr, Wg, att_src, att_dst,
           bg, W4, b4, g1, be1, g2, be2, g3, be3, g4, be4, Wfc, bfc):
    src0, dst0 = edge_index[0], edge_index[1]
    loop = jnp.arange(_N, dtype=edge_index.dtype)
    src = jnp.concatenate([src0, loop])
    dst = jnp.concatenate([dst0, loop])

    h0 = _mm(x, W1)
    deg = jnp.zeros((_N,), jnp.float32).at[dst].add(1.0)
    dis = lax.rsqrt(deg)[:, None]
    hs0 = _rowscale(h0, dis)
    agg1 = jnp.zeros((_N, 64), jnp.float32).at[dst].add(hs0[src])
    h1 = _bnrelu(_rowscale(agg1, dis), g1, be1)

    s = jnp.zeros((_N, 64), jnp.float32).at[dst0].add(h1[src0])
    c = jnp.zeros((_N,), jnp.float32).at[dst0].add(1.0)
    mean = _rowscale(s, 1.0 / jnp.maximum(c, 1.0)[:, None])
    h2 = _bnrelu(_mm2(mean, Wl, h1, Wr), g2, be2)

    hg = _mm(h2, Wg)
    eye4 = jnp.eye(4, dtype=jnp.float32)
    A_src = (att_src[:, :, None] * eye4[:, None, :]).reshape(1024, 4)
    A_dst = (att_dst[:, :, None] * eye4[:, None, :]).reshape(1024, 4)
    asrc = _mm(hg, A_src)
    adst = _mm(hg, A_dst)
    e = asrc[src] + adst[dst]
    e = jnp.where(e > 0, e, 0.2 * e)
    emax = jax.ops.segment_max(e, dst, num_segments=_N)
    ex = jnp.exp(e - emax[dst])
    den = jax.ops.segment_sum(ex, dst, num_segments=_N)
    alpha = ex / (den[dst] + 1e-16)
    hr = hg.reshape(_N, 4, 256)
    agg3 = jax.ops.segment_sum(hr[src] * alpha[:, :, None], dst,
                               num_segments=_N)
    agg3 = agg3.reshape(_N, 1024)

    Mh = jnp.tile(jnp.eye(256, dtype=jnp.float32), (4, 1)) * 0.25
    h3 = _bnrelu(_mm(agg3, Mh), g3, be3)
    hs3 = _rowscale(_mm(h3, W4), dis)
    agg4 = jnp.zeros((_N, 512), jnp.float32).at[dst].add(hs3[src])
    h4 = _bnrelu(_rowscale(agg4, dis), g4, be4)

    out = pl.pallas_call(
        _pool_body,
        out_shape=jax.ShapeDtypeStruct((_G, 1024), jnp.float32),
        scratch_shapes=[pltpu.VMEM((_G, 512), jnp.float32)],
    )(h4, batch[:, None], Wfc, bfc[None, :])
    return out
